# Initial kernel scaffold; baseline (speedup 1.0000x reference)
import jax
import jax.numpy as jnp
from jax.experimental import pallas as pl


def kernel(x, edge_index, W1, b1, W2, b2, Wc, bc):
    raise NotImplementedError("write your pallas kernel here")


# speedup vs baseline: 61.3584x; 61.3584x over previous
import functools

import jax
import jax.numpy as jnp
from jax import lax
from jax.experimental import pallas as pl
from jax.experimental.pallas import tpu as pltpu
from jax.experimental.pallas import tpu_sc as plsc

N_NODES = 100000
N_FEAT = 4
D = 8
N_OUT = 3

NC, NS = 2, 16
NW = NC * NS
N_PAD = 100352
ROWS_PER_SUB = N_PAD // NS

E_EDGES = 3200000
EB = 128
KB = 16
EPW = 100352
E_PAD = EPW * NW
NGROUPS = EPW // (EB * KB)
BATCHES_PER_W = EPW // EB

TC_BLK = 1024
TC_GRID = N_PAD // TC_BLK

_mesh = plsc.VectorSubcoreMesh(core_axis_name="c", subcore_axis_name="s")


@functools.partial(
    pl.kernel,
    out_type=jax.ShapeDtypeStruct((NC, N_PAD), jnp.float32),
    mesh=_mesh,
    scratch_types=[
        pltpu.VMEM((KB, EB), jnp.int32),
        pltpu.VMEM((EB,), jnp.float32),
        pltpu.VMEM((ROWS_PER_SUB,), jnp.float32),
        pltpu.VMEM_SHARED((N_PAD,), jnp.float32),
    ],
    compiler_params=pltpu.CompilerParams(use_tc_tiling_on_sc=False),
)
def _sc_degree(dst_hbm, out_hbm, dstv, onesv, zbuf, acc):
    c = lax.axis_index("c")
    s = lax.axis_index("s")
    wid = c * NS + s
    rs = s * ROWS_PER_SUB

    def zstep(i, carry):
        zbuf[pl.ds(i * 16, 16)] = jnp.zeros((16,), jnp.float32)
        return carry

    lax.fori_loop(0, ROWS_PER_SUB // 16, zstep, 0)
    for i in range(EB // 16):
        onesv[pl.ds(i * 16, 16)] = jnp.ones((16,), jnp.float32)
    pltpu.sync_copy(zbuf, acc.at[pl.ds(rs, ROWS_PER_SUB)])
    plsc.subcore_barrier()

    base_row = wid * BATCHES_PER_W

    def group(g, carry):
        pltpu.sync_copy(dst_hbm.at[pl.ds(base_row + g * KB, KB), :], dstv)
        for j in range(KB):
            pltpu.sync_copy(onesv, acc.at[dstv.at[j]], add=True)
        return carry

    lax.fori_loop(0, NGROUPS, group, 0)
    plsc.subcore_barrier()
    pltpu.sync_copy(acc.at[pl.ds(rs, ROWS_PER_SUB)],
                    out_hbm.at[c, pl.ds(rs, ROWS_PER_SUB)])


@functools.partial(
    pl.kernel,
    out_type=jax.ShapeDtypeStruct((NC, N_PAD, D), jnp.float32),
    mesh=_mesh,
    scratch_types=[
        pltpu.VMEM((KB, EB), jnp.int32),
        pltpu.VMEM((KB, EB), jnp.int32),
        pltpu.VMEM((KB * EB, D), jnp.float32),
        pltpu.VMEM_SHARED((N_PAD, D), jnp.float32),
        pltpu.SemaphoreType.DMA,
    ],
    compiler_params=pltpu.CompilerParams(use_tc_tiling_on_sc=False),
)
def _sc_aggregate(hs_hbm, src_hbm, dst_hbm, out_hbm, srcv, dstv, rowsv, acc, sem):
    c = lax.axis_index("c")
    s = lax.axis_index("s")
    wid = c * NS + s
    rs = s * ROWS_PER_SUB

    pltpu.sync_copy(hs_hbm.at[pl.ds(rs, ROWS_PER_SUB), :],
                    acc.at[pl.ds(rs, ROWS_PER_SUB), :])
    plsc.subcore_barrier()

    base_row = wid * BATCHES_PER_W

    def group(g, carry):
        r0 = base_row + g * KB
        pltpu.sync_copy(src_hbm.at[pl.ds(r0, KB), :], srcv)
        pltpu.sync_copy(dst_hbm.at[pl.ds(r0, KB), :], dstv)
        for j in range(KB):
            pltpu.async_copy(hs_hbm.at[srcv.at[j]],
                             rowsv.at[pl.ds(j * EB, EB), :], sem)
        for j in range(KB):
            pltpu.make_async_copy(hs_hbm.at[srcv.at[j]],
                                  rowsv.at[pl.ds(j * EB, EB), :], sem).wait()
        for j in range(KB):
            pltpu.sync_copy(rowsv.at[pl.ds(j * EB, EB), :],
                            acc.at[dstv.at[j]], add=True)
        return carry

    lax.fori_loop(0, NGROUPS, group, 0)
    plsc.subcore_barrier()
    pltpu.sync_copy(acc.at[pl.ds(rs, ROWS_PER_SUB), :],
                    out_hbm.at[c, pl.ds(rs, ROWS_PER_SUB), :])


def _tc_prep_body(degp_ref, x_ref, w1_ref, dinv_ref, hs_ref):
    deg = degp_ref[0, :] + degp_ref[1, :] + 1.0
    dinv = 1.0 / jnp.sqrt(deg)
    h = jnp.dot(x_ref[...], w1_ref[...], preferred_element_type=jnp.float32)
    dinv_ref[...] = dinv[:, None]
    hs_ref[...] = h * dinv[:, None]


def _tc_prep(degp, x_pad, w1p):
    return pl.pallas_call(
        _tc_prep_body,
        grid=(TC_GRID,),
        in_specs=[
            pl.BlockSpec((NC, TC_BLK), lambda i: (0, i)),
            pl.BlockSpec((TC_BLK, N_FEAT), lambda i: (i, 0)),
            pl.BlockSpec((N_FEAT, D), lambda i: (0, 0)),
        ],
        out_specs=[
            pl.BlockSpec((TC_BLK, 1), lambda i: (i, 0)),
            pl.BlockSpec((TC_BLK, D), lambda i: (i, 0)),
        ],
        out_shape=[
            jax.ShapeDtypeStruct((N_PAD, 1), jnp.float32),
            jax.ShapeDtypeStruct((N_PAD, D), jnp.float32),
        ],
    )(degp, x_pad, w1p)


def _tc_mid_body(dinv_ref, p_ref, hs_ref, w2_ref, b1_ref, out_ref):
    dinv = dinv_ref[...]
    agg = p_ref[0] + p_ref[1] - hs_ref[...]
    h1 = jnp.maximum(agg * dinv + b1_ref[...], 0.0)
    out_ref[...] = jnp.dot(h1, w2_ref[...],
                           preferred_element_type=jnp.float32) * dinv


def _tc_mid(dinv, p, hs1, w2p, b1p):
    return pl.pallas_call(
        _tc_mid_body,
        grid=(TC_GRID,),
        in_specs=[
            pl.BlockSpec((TC_BLK, 1), lambda i: (i, 0)),
            pl.BlockSpec((NC, TC_BLK, D), lambda i: (0, i, 0)),
            pl.BlockSpec((TC_BLK, D), lambda i: (i, 0)),
            pl.BlockSpec((D, D), lambda i: (0, 0)),
            pl.BlockSpec((1, D), lambda i: (0, 0)),
        ],
        out_specs=pl.BlockSpec((TC_BLK, D), lambda i: (i, 0)),
        out_shape=jax.ShapeDtypeStruct((N_PAD, D), jnp.float32),
    )(dinv, p, hs1, w2p, b1p)


def _tc_final_body(dinv_ref, p_ref, hs_ref, wc_ref, b2_ref, bc_ref, out_ref):
    dinv = dinv_ref[...]
    agg = p_ref[0] + p_ref[1] - hs_ref[...]
    h2 = jnp.maximum(agg * dinv + b2_ref[...], 0.0)
    out_ref[...] = jnp.dot(h2, wc_ref[...],
                           preferred_element_type=jnp.float32) + bc_ref[...]


def _tc_final(dinv, p, hs2, wcp, b2p, bcp):
    return pl.pallas_call(
        _tc_final_body,
        grid=(TC_GRID,),
        in_specs=[
            pl.BlockSpec((TC_BLK, 1), lambda i: (i, 0)),
            pl.BlockSpec((NC, TC_BLK, D), lambda i: (0, i, 0)),
            pl.BlockSpec((TC_BLK, D), lambda i: (i, 0)),
            pl.BlockSpec((D, N_OUT), lambda i: (0, 0)),
            pl.BlockSpec((1, D), lambda i: (0, 0)),
            pl.BlockSpec((1, N_OUT), lambda i: (0, 0)),
        ],
        out_specs=pl.BlockSpec((TC_BLK, N_OUT), lambda i: (i, 0)),
        out_shape=jax.ShapeDtypeStruct((N_PAD, N_OUT), jnp.float32),
    )(dinv, p, hs2, wcp, b2p, bcp)


def kernel(x, edge_index, W1, b1, W2, b2, Wc, bc):
    src = edge_index[0].astype(jnp.int32)
    dst = edge_index[1].astype(jnp.int32)
    n_extra = E_PAD - E_EDGES
    pad_idx = N_NODES + (jnp.arange(n_extra, dtype=jnp.int32) % 256)
    src_p = jnp.concatenate([src, pad_idx]).reshape(E_PAD // EB, EB)
    dst_p = jnp.concatenate([dst, pad_idx]).reshape(E_PAD // EB, EB)

    x_pad = jnp.zeros((N_PAD, N_FEAT), jnp.float32).at[:N_NODES].set(x)
    w1p = jnp.zeros((N_FEAT, D), jnp.float32).at[:, :6].set(W1)
    w2p = jnp.zeros((D, D), jnp.float32).at[:6, :6].set(W2)
    wcp = jnp.zeros((D, N_OUT), jnp.float32).at[:6, :].set(Wc)
    b1p = jnp.zeros((1, D), jnp.float32).at[0, :6].set(b1)
    b2p = jnp.zeros((1, D), jnp.float32).at[0, :6].set(b2)
    bcp = bc[None, :]

    degp = _sc_degree(dst_p)
    dinv, hs1 = _tc_prep(degp, x_pad, w1p)
    p1 = _sc_aggregate(hs1, src_p, dst_p)
    hs2 = _tc_mid(dinv, p1, hs1, w2p, b1p)
    p2 = _sc_aggregate(hs2, src_p, dst_p)
    out = _tc_final(dinv, p2, hs2, wcp, b2p, bcp)
    return out[:N_NODES]


# speedup vs baseline: 80.6021x; 1.3136x over previous
import functools

import jax
import jax.numpy as jnp
from jax import lax
from jax.experimental import pallas as pl
from jax.experimental.pallas import tpu as pltpu
from jax.experimental.pallas import tpu_sc as plsc

N_NODES = 100000
N_FEAT = 4
D = 8
N_OUT = 3

NC, NS = 2, 16
NW = NC * NS
N_PAD = 100352
ROWS_PER_SUB = N_PAD // NS

E_EDGES = 3200000
EB = 128
KB = 14
EPW = 100352
E_PAD = EPW * NW
NGROUPS = EPW // (EB * KB)
NPAIRS = NGROUPS // 2
BATCHES_PER_W = EPW // EB
IDX_ROWS = E_PAD // EB + 2 * KB

TC_BLK = 1024
TC_GRID = N_PAD // TC_BLK

_mesh = plsc.VectorSubcoreMesh(core_axis_name="c", subcore_axis_name="s")


@functools.partial(
    pl.kernel,
    out_type=jax.ShapeDtypeStruct((NC, N_PAD), jnp.float32),
    mesh=_mesh,
    scratch_types=[
        pltpu.VMEM((KB, EB), jnp.int32),
        pltpu.VMEM((KB, EB), jnp.int32),
        pltpu.VMEM((EB,), jnp.float32),
        pltpu.VMEM((ROWS_PER_SUB,), jnp.float32),
        pltpu.VMEM_SHARED((N_PAD,), jnp.float32),
        pltpu.SemaphoreType.DMA,
        pltpu.SemaphoreType.DMA,
    ],
    compiler_params=pltpu.CompilerParams(use_tc_tiling_on_sc=False),
)
def _sc_degree(dst_hbm, out_hbm, dstv0, dstv1, onesv, zbuf, acc, isem0, isem1):
    c = lax.axis_index("c")
    s = lax.axis_index("s")
    wid = c * NS + s
    rs = s * ROWS_PER_SUB

    def zstep(i, carry):
        zbuf[pl.ds(i * 16, 16)] = jnp.zeros((16,), jnp.float32)
        return carry

    lax.fori_loop(0, ROWS_PER_SUB // 16, zstep, 0)
    for i in range(EB // 16):
        onesv[pl.ds(i * 16, 16)] = jnp.ones((16,), jnp.float32)
    pltpu.sync_copy(zbuf, acc.at[pl.ds(rs, ROWS_PER_SUB)])
    plsc.subcore_barrier()

    base_row = wid * BATCHES_PER_W

    def load(g, buf, sem):
        pltpu.async_copy(dst_hbm.at[pl.ds(base_row + g * KB, KB), :], buf, sem)

    def wait(g, buf, sem):
        pltpu.make_async_copy(dst_hbm.at[pl.ds(base_row + g * KB, KB), :],
                              buf, sem).wait()

    def scat(buf):
        for j in range(KB):
            pltpu.sync_copy(onesv, acc.at[buf.at[j]], add=True)

    load(0, dstv0, isem0)
    wait(0, dstv0, isem0)
    load(1, dstv1, isem1)

    def pair(i, carry):
        g0 = 2 * i
        scat(dstv0)
        load(g0 + 2, dstv0, isem0)
        wait(g0 + 1, dstv1, isem1)
        scat(dstv1)
        load(g0 + 3, dstv1, isem1)
        wait(g0 + 2, dstv0, isem0)
        return carry

    lax.fori_loop(0, NPAIRS, pair, 0)
    wait(2 * NPAIRS + 1, dstv1, isem1)
    plsc.subcore_barrier()
    pltpu.sync_copy(acc.at[pl.ds(rs, ROWS_PER_SUB)],
                    out_hbm.at[c, pl.ds(rs, ROWS_PER_SUB)])


@functools.partial(
    pl.kernel,
    out_type=jax.ShapeDtypeStruct((NC, N_PAD, D), jnp.float32),
    mesh=_mesh,
    scratch_types=[
        pltpu.VMEM((KB, EB), jnp.int32),
        pltpu.VMEM((KB, EB), jnp.int32),
        pltpu.VMEM((KB, EB), jnp.int32),
        pltpu.VMEM((KB, EB), jnp.int32),
        pltpu.VMEM((KB * EB, D), jnp.float32),
        pltpu.VMEM((KB * EB, D), jnp.float32),
        pltpu.VMEM_SHARED((N_PAD, D), jnp.float32),
        pltpu.SemaphoreType.DMA,
        pltpu.SemaphoreType.DMA,
        pltpu.SemaphoreType.DMA,
        pltpu.SemaphoreType.DMA,
    ],
    compiler_params=pltpu.CompilerParams(use_tc_tiling_on_sc=False),
)
def _sc_aggregate(hs_hbm, src_hbm, dst_hbm, out_hbm,
                  srcv0, srcv1, dstv0, dstv1, rowsv0, rowsv1, acc,
                  isem0, isem1, gsem0, gsem1):
    c = lax.axis_index("c")
    s = lax.axis_index("s")
    wid = c * NS + s
    rs = s * ROWS_PER_SUB

    pltpu.sync_copy(hs_hbm.at[pl.ds(rs, ROWS_PER_SUB), :],
                    acc.at[pl.ds(rs, ROWS_PER_SUB), :])
    plsc.subcore_barrier()

    base_row = wid * BATCHES_PER_W

    def load_idx(g, sbuf, dbuf, sem):
        r0 = base_row + g * KB
        pltpu.async_copy(src_hbm.at[pl.ds(r0, KB), :], sbuf, sem)
        pltpu.async_copy(dst_hbm.at[pl.ds(r0, KB), :], dbuf, sem)

    def wait_idx(g, sbuf, dbuf, sem):
        r0 = base_row + g * KB
        pltpu.make_async_copy(src_hbm.at[pl.ds(r0, KB), :], sbuf, sem).wait()
        pltpu.make_async_copy(dst_hbm.at[pl.ds(r0, KB), :], dbuf, sem).wait()

    def fire_gathers(sbuf, rbuf, sem):
        for j in range(KB):
            pltpu.async_copy(hs_hbm.at[sbuf.at[j]],
                             rbuf.at[pl.ds(j * EB, EB), :], sem)

    def drain_gathers(sbuf, rbuf, sem):
        for j in range(KB):
            pltpu.make_async_copy(hs_hbm.at[sbuf.at[j]],
                                  rbuf.at[pl.ds(j * EB, EB), :], sem).wait()

    def scat(dbuf, rbuf):
        for j in range(KB):
            pltpu.sync_copy(rbuf.at[pl.ds(j * EB, EB), :],
                            acc.at[dbuf.at[j]], add=True)

    load_idx(0, srcv0, dstv0, isem0)
    wait_idx(0, srcv0, dstv0, isem0)
    fire_gathers(srcv0, rowsv0, gsem0)
    load_idx(1, srcv1, dstv1, isem1)

    def pair(i, carry):
        g0 = 2 * i
        wait_idx(g0 + 1, srcv1, dstv1, isem1)
        fire_gathers(srcv1, rowsv1, gsem1)
        drain_gathers(srcv0, rowsv0, gsem0)
        scat(dstv0, rowsv0)
        load_idx(g0 + 2, srcv0, dstv0, isem0)
        wait_idx(g0 + 2, srcv0, dstv0, isem0)
        fire_gathers(srcv0, rowsv0, gsem0)
        drain_gathers(srcv1, rowsv1, gsem1)
        scat(dstv1, rowsv1)
        load_idx(g0 + 3, srcv1, dstv1, isem1)
        return carry

    lax.fori_loop(0, NPAIRS, pair, 0)
    drain_gathers(srcv0, rowsv0, gsem0)
    wait_idx(2 * NPAIRS + 1, srcv1, dstv1, isem1)
    plsc.subcore_barrier()
    pltpu.sync_copy(acc.at[pl.ds(rs, ROWS_PER_SUB), :],
                    out_hbm.at[c, pl.ds(rs, ROWS_PER_SUB), :])


def _tc_prep_body(degp_ref, x_ref, w1_ref, dinv_ref, hs_ref):
    deg = degp_ref[0, :] + degp_ref[1, :] + 1.0
    dinv = 1.0 / jnp.sqrt(deg)
    h = jnp.dot(x_ref[...], w1_ref[...], preferred_element_type=jnp.float32)
    dinv_ref[...] = dinv[:, None]
    hs_ref[...] = h * dinv[:, None]


def _tc_prep(degp, x_pad, w1p):
    return pl.pallas_call(
        _tc_prep_body,
        grid=(TC_GRID,),
        in_specs=[
            pl.BlockSpec((NC, TC_BLK), lambda i: (0, i)),
            pl.BlockSpec((TC_BLK, N_FEAT), lambda i: (i, 0)),
            pl.BlockSpec((N_FEAT, D), lambda i: (0, 0)),
        ],
        out_specs=[
            pl.BlockSpec((TC_BLK, 1), lambda i: (i, 0)),
            pl.BlockSpec((TC_BLK, D), lambda i: (i, 0)),
        ],
        out_shape=[
            jax.ShapeDtypeStruct((N_PAD, 1), jnp.float32),
            jax.ShapeDtypeStruct((N_PAD, D), jnp.float32),
        ],
    )(degp, x_pad, w1p)


def _tc_mid_body(dinv_ref, p_ref, hs_ref, w2_ref, b1_ref, out_ref):
    dinv = dinv_ref[...]
    agg = p_ref[0] + p_ref[1] - hs_ref[...]
    h1 = jnp.maximum(agg * dinv + b1_ref[...], 0.0)
    out_ref[...] = jnp.dot(h1, w2_ref[...],
                           preferred_element_type=jnp.float32) * dinv


def _tc_mid(dinv, p, hs1, w2p, b1p):
    return pl.pallas_call(
        _tc_mid_body,
        grid=(TC_GRID,),
        in_specs=[
            pl.BlockSpec((TC_BLK, 1), lambda i: (i, 0)),
            pl.BlockSpec((NC, TC_BLK, D), lambda i: (0, i, 0)),
            pl.BlockSpec((TC_BLK, D), lambda i: (i, 0)),
            pl.BlockSpec((D, D), lambda i: (0, 0)),
            pl.BlockSpec((1, D), lambda i: (0, 0)),
        ],
        out_specs=pl.BlockSpec((TC_BLK, D), lambda i: (i, 0)),
        out_shape=jax.ShapeDtypeStruct((N_PAD, D), jnp.float32),
    )(dinv, p, hs1, w2p, b1p)


def _tc_final_body(dinv_ref, p_ref, hs_ref, wc_ref, b2_ref, bc_ref, out_ref):
    dinv = dinv_ref[...]
    agg = p_ref[0] + p_ref[1] - hs_ref[...]
    h2 = jnp.maximum(agg * dinv + b2_ref[...], 0.0)
    out_ref[...] = jnp.dot(h2, wc_ref[...],
                           preferred_element_type=jnp.float32) + bc_ref[...]


def _tc_final(dinv, p, hs2, wcp, b2p, bcp):
    return pl.pallas_call(
        _tc_final_body,
        grid=(TC_GRID,),
        in_specs=[
            pl.BlockSpec((TC_BLK, 1), lambda i: (i, 0)),
            pl.BlockSpec((NC, TC_BLK, D), lambda i: (0, i, 0)),
            pl.BlockSpec((TC_BLK, D), lambda i: (i, 0)),
            pl.BlockSpec((D, N_OUT), lambda i: (0, 0)),
            pl.BlockSpec((1, D), lambda i: (0, 0)),
            pl.BlockSpec((1, N_OUT), lambda i: (0, 0)),
        ],
        out_specs=pl.BlockSpec((TC_BLK, N_OUT), lambda i: (i, 0)),
        out_shape=jax.ShapeDtypeStruct((N_PAD, N_OUT), jnp.float32),
    )(dinv, p, hs2, wcp, b2p, bcp)


def kernel(x, edge_index, W1, b1, W2, b2, Wc, bc):
    src = edge_index[0].astype(jnp.int32)
    dst = edge_index[1].astype(jnp.int32)
    n_extra = IDX_ROWS * EB - E_EDGES
    pad_idx = N_NODES + (jnp.arange(n_extra, dtype=jnp.int32) % 256)
    src_p = jnp.concatenate([src, pad_idx]).reshape(IDX_ROWS, EB)
    dst_p = jnp.concatenate([dst, pad_idx]).reshape(IDX_ROWS, EB)

    x_pad = jnp.zeros((N_PAD, N_FEAT), jnp.float32).at[:N_NODES].set(x)
    w1p = jnp.zeros((N_FEAT, D), jnp.float32).at[:, :6].set(W1)
    w2p = jnp.zeros((D, D), jnp.float32).at[:6, :6].set(W2)
    wcp = jnp.zeros((D, N_OUT), jnp.float32).at[:6, :].set(Wc)
    b1p = jnp.zeros((1, D), jnp.float32).at[0, :6].set(b1)
    b2p = jnp.zeros((1, D), jnp.float32).at[0, :6].set(b2)
    bcp = bc[None, :]

    degp = _sc_degree(dst_p)
    dinv, hs1 = _tc_prep(degp, x_pad, w1p)
    p1 = _sc_aggregate(hs1, src_p, dst_p)
    hs2 = _tc_mid(dinv, p1, hs1, w2p, b1p)
    p2 = _sc_aggregate(hs2, src_p, dst_p)
    out = _tc_final(dinv, p2, hs2, wcp, b2p, bcp)
    return out[:N_NODES]


# speedup vs baseline: 93.5907x; 1.1611x over previous
import functools

import jax
import jax.numpy as jnp
from jax import lax
from jax.experimental import pallas as pl
from jax.experimental.pallas import tpu as pltpu
from jax.experimental.pallas import tpu_sc as plsc

N_NODES = 100000
N_FEAT = 4
D = 8
N_OUT = 3

NC, NS = 2, 16
NW = NC * NS
N_PAD = 100352
ROWS_PER_SUB = N_PAD // NS
ROWS_PER_W = N_PAD // NW
CH = 784
NCH = ROWS_PER_SUB // CH

E_EDGES = 3200000
EB = 128
KB = 8
EPW = 100352
E_PAD = EPW * NW
NGROUPS = EPW // (EB * KB)
NPAIRS = NGROUPS // 2
BATCHES_PER_W = EPW // EB
IDX_ROWS = E_PAD // EB + 2 * KB

_mesh = plsc.VectorSubcoreMesh(core_axis_name="c", subcore_axis_name="s")
_f32 = jnp.float32


def _rsqrt16(x):
    i = plsc.bitcast(x, jnp.int32)
    i = 0x5F3759DF - lax.shift_right_logical(i, 1)
    y = plsc.bitcast(i, _f32)
    for _ in range(3):
        y = y * (1.5 - 0.5 * x * y * y)
    return y


def _iotas():
    iota = lax.iota(jnp.int32, 16)
    col = jnp.bitwise_and(iota, 7)
    half = lax.shift_right_logical(iota, 3)
    return iota, col, half


def _diag_matmul(tile, row_idx, wdt, iota):
    acc = None
    for s in range(8):
        cidx = jnp.bitwise_and(iota + s, 7)
        term = plsc.load_gather(tile, [row_idx, cidx]) * wdt[s]
        acc = term if acc is None else acc + term
    return acc


@functools.partial(
    pl.kernel,
    out_type=jax.ShapeDtypeStruct((NC, N_PAD), _f32),
    mesh=_mesh,
    scratch_types=[
        pltpu.VMEM((KB, EB), jnp.int32),
        pltpu.VMEM((KB, EB), jnp.int32),
        pltpu.VMEM((EB,), _f32),
        pltpu.VMEM((ROWS_PER_SUB,), _f32),
        pltpu.VMEM_SHARED((N_PAD,), _f32),
        pltpu.SemaphoreType.DMA,
        pltpu.SemaphoreType.DMA,
    ],
    compiler_params=pltpu.CompilerParams(use_tc_tiling_on_sc=False, needs_layout_passes=False),
)
def _sc_degree(dst_hbm, out_hbm, dstv0, dstv1, onesv, zbuf, acc, isem0, isem1):
    c = lax.axis_index("c")
    s = lax.axis_index("s")
    wid = c * NS + s
    rs = s * ROWS_PER_SUB

    def zstep(i, carry):
        zbuf[pl.ds(i * 16, 16)] = jnp.zeros((16,), _f32)
        return carry

    lax.fori_loop(0, ROWS_PER_SUB // 16, zstep, 0)
    for i in range(EB // 16):
        onesv[pl.ds(i * 16, 16)] = jnp.ones((16,), _f32)
    pltpu.sync_copy(zbuf, acc.at[pl.ds(rs, ROWS_PER_SUB)])
    plsc.subcore_barrier()

    base_row = wid * BATCHES_PER_W

    def load(g, buf, sem):
        pltpu.async_copy(dst_hbm.at[pl.ds(base_row + g * KB, KB), :], buf, sem)

    def wait(g, buf, sem):
        pltpu.make_async_copy(dst_hbm.at[pl.ds(base_row + g * KB, KB), :],
                              buf, sem).wait()

    def scat(buf):
        for j in range(KB):
            pltpu.sync_copy(onesv, acc.at[buf.at[j]], add=True)

    load(0, dstv0, isem0)
    wait(0, dstv0, isem0)
    load(1, dstv1, isem1)

    def pair(i, carry):
        g0 = 2 * i
        scat(dstv0)
        load(g0 + 2, dstv0, isem0)
        wait(g0 + 1, dstv1, isem1)
        scat(dstv1)
        load(g0 + 3, dstv1, isem1)
        wait(g0 + 2, dstv0, isem0)
        return carry

    lax.fori_loop(0, NPAIRS, pair, 0)
    wait(2 * NPAIRS + 1, dstv1, isem1)
    plsc.subcore_barrier()
    pltpu.sync_copy(acc.at[pl.ds(rs, ROWS_PER_SUB)],
                    out_hbm.at[c, pl.ds(rs, ROWS_PER_SUB)])


def _agg_pipeline(table, src_hbm, dst_hbm, acc,
                  srcv0, srcv1, dstv0, dstv1, rowsv0, rowsv1,
                  isem0, isem1, gsem0, gsem1, wid):
    base_row = wid * BATCHES_PER_W

    def load_idx(g, sbuf, dbuf, sem):
        r0 = base_row + g * KB
        pltpu.async_copy(src_hbm.at[pl.ds(r0, KB), :], sbuf, sem)
        pltpu.async_copy(dst_hbm.at[pl.ds(r0, KB), :], dbuf, sem)

    def wait_idx(g, sbuf, dbuf, sem):
        r0 = base_row + g * KB
        pltpu.make_async_copy(src_hbm.at[pl.ds(r0, KB), :], sbuf, sem).wait()
        pltpu.make_async_copy(dst_hbm.at[pl.ds(r0, KB), :], dbuf, sem).wait()

    def fire_gathers(sbuf, rbuf, sem):
        for j in range(KB):
            pltpu.async_copy(table.at[sbuf.at[j]],
                             rbuf.at[pl.ds(j * EB, EB), :], sem)

    def drain_gathers(sbuf, rbuf, sem):
        for j in range(KB):
            pltpu.make_async_copy(table.at[sbuf.at[j]],
                                  rbuf.at[pl.ds(j * EB, EB), :], sem).wait()

    def scat(dbuf, rbuf):
        for j in range(KB):
            pltpu.sync_copy(rbuf.at[pl.ds(j * EB, EB), :],
                            acc.at[dbuf.at[j]], add=True)

    load_idx(0, srcv0, dstv0, isem0)
    wait_idx(0, srcv0, dstv0, isem0)
    fire_gathers(srcv0, rowsv0, gsem0)
    load_idx(1, srcv1, dstv1, isem1)

    def pair(i, carry):
        g0 = 2 * i
        wait_idx(g0 + 1, srcv1, dstv1, isem1)
        fire_gathers(srcv1, rowsv1, gsem1)
        drain_gathers(srcv0, rowsv0, gsem0)
        scat(dstv0, rowsv0)
        load_idx(g0 + 2, srcv0, dstv0, isem0)
        wait_idx(g0 + 2, srcv0, dstv0, isem0)
        fire_gathers(srcv0, rowsv0, gsem0)
        drain_gathers(srcv1, rowsv1, gsem1)
        scat(dstv1, rowsv1)
        load_idx(g0 + 3, srcv1, dstv1, isem1)
        return carry

    lax.fori_loop(0, NPAIRS, pair, 0)
    drain_gathers(srcv0, rowsv0, gsem0)
    wait_idx(2 * NPAIRS + 1, srcv1, dstv1, isem1)


_AGG_SCRATCH = [
    pltpu.VMEM((KB, EB), jnp.int32),
    pltpu.VMEM((KB, EB), jnp.int32),
    pltpu.VMEM((KB, EB), jnp.int32),
    pltpu.VMEM((KB, EB), jnp.int32),
    pltpu.VMEM((KB * EB, D), _f32),
    pltpu.VMEM((KB * EB, D), _f32),
    pltpu.VMEM_SHARED((N_PAD, D), _f32),
    pltpu.SemaphoreType.DMA,
    pltpu.SemaphoreType.DMA,
    pltpu.SemaphoreType.DMA,
    pltpu.SemaphoreType.DMA,
]


@functools.partial(
    pl.kernel,
    out_type=(
        jax.ShapeDtypeStruct((NC, N_PAD, D), _f32),
        jax.ShapeDtypeStruct((NC, N_PAD, D), _f32),
    ),
    mesh=_mesh,
    scratch_types=_AGG_SCRATCH + [
        pltpu.VMEM((CH, D), _f32),
        pltpu.VMEM((CH, D), _f32),
        pltpu.VMEM((CH,), _f32),
        pltpu.VMEM((CH,), _f32),
        pltpu.VMEM((D, 16), _f32),
        pltpu.VMEM((16,), _f32),
    ],
    compiler_params=pltpu.CompilerParams(use_tc_tiling_on_sc=False, needs_layout_passes=False),
)
def _sc_prep_agg1(degp_hbm, x8_hbm, src_hbm, dst_hbm, w1d_hbm,
                  p1_hbm, hs1_hbm,
                  srcv0, srcv1, dstv0, dstv1, rowsv0, rowsv1, acc,
                  isem0, isem1, gsem0, gsem1,
                  xt, ht, pd0, pd1, wdt, dvb):
    c = lax.axis_index("c")
    s = lax.axis_index("s")
    wid = c * NS + s
    rs = s * ROWS_PER_SUB
    iota, col, half = _iotas()

    pltpu.sync_copy(w1d_hbm, wdt)
    for ch in range(NCH):
        r0 = rs + ch * CH
        pltpu.sync_copy(x8_hbm.at[pl.ds(r0, CH), :], xt)
        pltpu.sync_copy(degp_hbm.at[0, pl.ds(r0, CH)], pd0)
        pltpu.sync_copy(degp_hbm.at[1, pl.ds(r0, CH)], pd1)

        def grp(g, carry):
            deg = pd0[pl.ds(g * 16, 16)] + pd1[pl.ds(g * 16, 16)] + 1.0
            dvb[...] = _rsqrt16(deg)
            for k in range(8):
                ridx = g * 16 + 2 * k + half
                h = _diag_matmul(xt, ridx, wdt, iota)
                dv8 = plsc.load_gather(dvb, [2 * k + half])
                plsc.store_scatter(ht, [ridx, col], h * dv8)
            return carry

        lax.fori_loop(0, CH // 16, grp, 0)
        pltpu.sync_copy(ht, hs1_hbm.at[c, pl.ds(r0, CH), :])
        pltpu.sync_copy(ht, acc.at[pl.ds(r0, CH), :])

    plsc.subcore_barrier()
    _agg_pipeline(hs1_hbm.at[c], src_hbm, dst_hbm, acc,
                  srcv0, srcv1, dstv0, dstv1, rowsv0, rowsv1,
                  isem0, isem1, gsem0, gsem1, wid)
    plsc.subcore_barrier()
    pltpu.sync_copy(acc.at[pl.ds(rs, ROWS_PER_SUB), :],
                    p1_hbm.at[c, pl.ds(rs, ROWS_PER_SUB), :])


@functools.partial(
    pl.kernel,
    out_type=(
        jax.ShapeDtypeStruct((NC, N_PAD, D), _f32),
        jax.ShapeDtypeStruct((NC, N_PAD, D), _f32),
    ),
    mesh=_mesh,
    scratch_types=_AGG_SCRATCH + [
        pltpu.VMEM((CH, D), _f32),
        pltpu.VMEM((CH, D), _f32),
        pltpu.VMEM((CH, D), _f32),
        pltpu.VMEM((CH, D), _f32),
        pltpu.VMEM((CH,), _f32),
        pltpu.VMEM((CH,), _f32),
        pltpu.VMEM((D, 16), _f32),
        pltpu.VMEM((16,), _f32),
        pltpu.VMEM((16,), _f32),
    ],
    compiler_params=pltpu.CompilerParams(use_tc_tiling_on_sc=False, needs_layout_passes=False),
)
def _sc_mid_agg2(degp_hbm, p1_hbm, hs1_hbm, src_hbm, dst_hbm, w2d_hbm, b1_hbm,
                 p2_hbm, hs2_hbm,
                 srcv0, srcv1, dstv0, dstv1, rowsv0, rowsv1, acc,
                 isem0, isem1, gsem0, gsem1,
                 at0, at1, hst, ht, pd0, pd1, wdt, dvb, bt):
    c = lax.axis_index("c")
    s = lax.axis_index("s")
    wid = c * NS + s
    rs = s * ROWS_PER_SUB
    iota, col, half = _iotas()

    pltpu.sync_copy(w2d_hbm, wdt)
    pltpu.sync_copy(b1_hbm, bt)
    for ch in range(NCH):
        r0 = rs + ch * CH
        pltpu.sync_copy(p1_hbm.at[0, pl.ds(r0, CH), :], at0)
        pltpu.sync_copy(p1_hbm.at[1, pl.ds(r0, CH), :], at1)
        pltpu.sync_copy(hs1_hbm.at[c, pl.ds(r0, CH), :], hst)
        pltpu.sync_copy(degp_hbm.at[0, pl.ds(r0, CH)], pd0)
        pltpu.sync_copy(degp_hbm.at[1, pl.ds(r0, CH)], pd1)

        def grp(g, carry):
            deg = pd0[pl.ds(g * 16, 16)] + pd1[pl.ds(g * 16, 16)] + 1.0
            dvb[...] = _rsqrt16(deg)
            bv = bt[...]
            for k in range(8):
                ridx = g * 16 + 2 * k + half
                agg = (plsc.load_gather(at0, [ridx, col])
                       + plsc.load_gather(at1, [ridx, col])
                       - plsc.load_gather(hst, [ridx, col]))
                dv8 = plsc.load_gather(dvb, [2 * k + half])
                h1 = jnp.maximum(agg * dv8 + bv, 0.0)
                plsc.store_scatter(at0, [ridx, col], h1)
            for k in range(8):
                ridx = g * 16 + 2 * k + half
                h2 = _diag_matmul(at0, ridx, wdt, iota)
                dv8 = plsc.load_gather(dvb, [2 * k + half])
                plsc.store_scatter(ht, [ridx, col], h2 * dv8)
            return carry

        lax.fori_loop(0, CH // 16, grp, 0)
        pltpu.sync_copy(ht, hs2_hbm.at[c, pl.ds(r0, CH), :])
        pltpu.sync_copy(ht, acc.at[pl.ds(r0, CH), :])

    plsc.subcore_barrier()
    _agg_pipeline(hs2_hbm.at[c], src_hbm, dst_hbm, acc,
                  srcv0, srcv1, dstv0, dstv1, rowsv0, rowsv1,
                  isem0, isem1, gsem0, gsem1, wid)
    plsc.subcore_barrier()
    pltpu.sync_copy(acc.at[pl.ds(rs, ROWS_PER_SUB), :],
                    p2_hbm.at[c, pl.ds(rs, ROWS_PER_SUB), :])


@functools.partial(
    pl.kernel,
    out_type=jax.ShapeDtypeStruct((N_PAD, D), _f32),
    mesh=_mesh,
    scratch_types=[
        pltpu.VMEM((CH, D), _f32),
        pltpu.VMEM((CH, D), _f32),
        pltpu.VMEM((CH, D), _f32),
        pltpu.VMEM((CH, D), _f32),
        pltpu.VMEM((CH,), _f32),
        pltpu.VMEM((CH,), _f32),
        pltpu.VMEM((D, 16), _f32),
        pltpu.VMEM((16,), _f32),
        pltpu.VMEM((16,), _f32),
        pltpu.VMEM((16,), _f32),
    ],
    compiler_params=pltpu.CompilerParams(use_tc_tiling_on_sc=False, needs_layout_passes=False),
)
def _sc_final(degp_hbm, p2_hbm, hs2_hbm, wcd_hbm, b2_hbm, bc_hbm, out_hbm,
              at0, at1, hst, ot, pd0, pd1, wdt, dvb, bt2, btc):
    c = lax.axis_index("c")
    s = lax.axis_index("s")
    wid = c * NS + s
    rw = wid * ROWS_PER_W
    iota, col, half = _iotas()

    pltpu.sync_copy(wcd_hbm, wdt)
    pltpu.sync_copy(b2_hbm, bt2)
    pltpu.sync_copy(bc_hbm, btc)
    for ch in range(ROWS_PER_W // CH):
        r0 = rw + ch * CH
        pltpu.sync_copy(p2_hbm.at[0, pl.ds(r0, CH), :], at0)
        pltpu.sync_copy(p2_hbm.at[1, pl.ds(r0, CH), :], at1)
        pltpu.sync_copy(hs2_hbm.at[c, pl.ds(r0, CH), :], hst)
        pltpu.sync_copy(degp_hbm.at[0, pl.ds(r0, CH)], pd0)
        pltpu.sync_copy(degp_hbm.at[1, pl.ds(r0, CH)], pd1)

        def grp(g, carry):
            deg = pd0[pl.ds(g * 16, 16)] + pd1[pl.ds(g * 16, 16)] + 1.0
            dvb[...] = _rsqrt16(deg)
            bv2 = bt2[...]
            bvc = btc[...]
            for k in range(8):
                ridx = g * 16 + 2 * k + half
                agg = (plsc.load_gather(at0, [ridx, col])
                       + plsc.load_gather(at1, [ridx, col])
                       - plsc.load_gather(hst, [ridx, col]))
                dv8 = plsc.load_gather(dvb, [2 * k + half])
                h2 = jnp.maximum(agg * dv8 + bv2, 0.0)
                plsc.store_scatter(at0, [ridx, col], h2)
            for k in range(8):
                ridx = g * 16 + 2 * k + half
                o = _diag_matmul(at0, ridx, wdt, iota) + bvc
                plsc.store_scatter(ot, [ridx, col], o)
            return carry

        lax.fori_loop(0, CH // 16, grp, 0)
        pltpu.sync_copy(ot, out_hbm.at[pl.ds(r0, CH), :])


def _diagonalize(wp):
    j = jnp.arange(16) % 8
    srange = jnp.arange(8)[:, None]
    return wp[(j[None, :] + srange) % 8, j[None, :]]


def kernel(x, edge_index, W1, b1, W2, b2, Wc, bc):
    src = edge_index[0].astype(jnp.int32)
    dst = edge_index[1].astype(jnp.int32)
    n_extra = IDX_ROWS * EB - E_EDGES
    pad_idx = N_NODES + (jnp.arange(n_extra, dtype=jnp.int32) % 256)
    src_p = jnp.concatenate([src, pad_idx]).reshape(IDX_ROWS, EB)
    dst_p = jnp.concatenate([dst, pad_idx]).reshape(IDX_ROWS, EB)

    x8 = jnp.zeros((N_PAD, D), _f32).at[:N_NODES, :N_FEAT].set(x)
    w1p = jnp.zeros((D, D), _f32).at[:N_FEAT, :6].set(W1)
    w2p = jnp.zeros((D, D), _f32).at[:6, :6].set(W2)
    wcp = jnp.zeros((D, D), _f32).at[:6, :N_OUT].set(Wc)
    w1d = _diagonalize(w1p)
    w2d = _diagonalize(w2p)
    wcd = _diagonalize(wcp)
    b1t = jnp.tile(jnp.zeros((D,), _f32).at[:6].set(b1), 2)
    b2t = jnp.tile(jnp.zeros((D,), _f32).at[:6].set(b2), 2)
    bct = jnp.tile(jnp.zeros((D,), _f32).at[:N_OUT].set(bc), 2)

    degp = _sc_degree(dst_p)
    p1, hs1 = _sc_prep_agg1(degp, x8, src_p, dst_p, w1d)
    p2, hs2 = _sc_mid_agg2(degp, p1, hs1, src_p, dst_p, w2d, b1t)
    out8 = _sc_final(degp, p2, hs2, wcd, b2t, bct)
    return out8[:N_NODES, :N_OUT]


# speedup vs baseline: 98.4308x; 1.0517x over previous
import functools

import jax
import jax.numpy as jnp
from jax import lax
from jax.experimental import pallas as pl
from jax.experimental.pallas import tpu as pltpu
from jax.experimental.pallas import tpu_sc as plsc

N_NODES = 100000
N_FEAT = 4
D = 8
N_OUT = 3

NC, NS = 2, 16
NW = NC * NS
N_PAD = 100352
ROWS_PER_SUB = N_PAD // NS
ROWS_PER_W = N_PAD // NW
CH = 784
NCH = ROWS_PER_SUB // CH

E_EDGES = 3200000
EB = 128
KB = 14
EPW = 100352
E_PAD = EPW * NW
NGROUPS = EPW // (EB * KB)
NPAIRS = NGROUPS // 2
BATCHES_PER_W = EPW // EB
IDX_ROWS = E_PAD // EB + 2 * KB

_mesh = plsc.VectorSubcoreMesh(core_axis_name="c", subcore_axis_name="s")
_f32 = jnp.float32


def _rsqrt16(x):
    i = plsc.bitcast(x, jnp.int32)
    i = 0x5F3759DF - lax.shift_right_logical(i, 1)
    y = jnp.abs(plsc.bitcast(i, _f32))
    for _ in range(6):
        y = y * (1.5 - 0.5 * x * y * y)
    return y


def _iotas():
    iota = lax.iota(jnp.int32, 16)
    col = jnp.bitwise_and(iota, 7)
    half = lax.shift_right_logical(iota, 3)
    return iota, col, half


def _diag_matmul(tile, row_idx, wdt, iota):
    acc = None
    for s in range(8):
        cidx = jnp.bitwise_and(iota + s, 7)
        term = plsc.load_gather(tile, [row_idx, cidx]) * wdt[s]
        acc = term if acc is None else acc + term
    return acc


@functools.partial(
    pl.kernel,
    out_type=jax.ShapeDtypeStruct((NC, N_PAD), _f32),
    mesh=_mesh,
    scratch_types=[
        pltpu.VMEM((KB, EB), jnp.int32),
        pltpu.VMEM((KB, EB), jnp.int32),
        pltpu.VMEM((EB,), _f32),
        pltpu.VMEM((ROWS_PER_SUB,), _f32),
        pltpu.VMEM_SHARED((N_PAD,), _f32),
        pltpu.SemaphoreType.DMA,
        pltpu.SemaphoreType.DMA,
    ],
    compiler_params=pltpu.CompilerParams(use_tc_tiling_on_sc=False, needs_layout_passes=False),
)
def _sc_degree(dst_hbm, out_hbm, dstv0, dstv1, onesv, zbuf, acc, isem0, isem1):
    c = lax.axis_index("c")
    s = lax.axis_index("s")
    wid = c * NS + s
    rs = s * ROWS_PER_SUB

    def zstep(i, carry):
        zbuf[pl.ds(i * 16, 16)] = jnp.zeros((16,), _f32)
        return carry

    lax.fori_loop(0, ROWS_PER_SUB // 16, zstep, 0)
    for i in range(EB // 16):
        onesv[pl.ds(i * 16, 16)] = jnp.ones((16,), _f32)
    pltpu.sync_copy(zbuf, acc.at[pl.ds(rs, ROWS_PER_SUB)])
    plsc.subcore_barrier()

    base_row = wid * BATCHES_PER_W

    def load(g, buf, sem):
        pltpu.async_copy(dst_hbm.at[pl.ds(base_row + g * KB, KB), :], buf, sem)

    def wait(g, buf, sem):
        pltpu.make_async_copy(dst_hbm.at[pl.ds(base_row + g * KB, KB), :],
                              buf, sem).wait()

    def scat(buf):
        for j in range(KB):
            pltpu.sync_copy(onesv, acc.at[buf.at[j]], add=True)

    load(0, dstv0, isem0)
    wait(0, dstv0, isem0)
    load(1, dstv1, isem1)

    def pair(i, carry):
        g0 = 2 * i
        scat(dstv0)
        load(g0 + 2, dstv0, isem0)
        wait(g0 + 1, dstv1, isem1)
        scat(dstv1)
        load(g0 + 3, dstv1, isem1)
        wait(g0 + 2, dstv0, isem0)
        return carry

    lax.fori_loop(0, NPAIRS, pair, 0)
    wait(2 * NPAIRS + 1, dstv1, isem1)
    plsc.subcore_barrier()
    pltpu.sync_copy(acc.at[pl.ds(rs, ROWS_PER_SUB)],
                    out_hbm.at[c, pl.ds(rs, ROWS_PER_SUB)])


def _agg_pipeline(table, src_hbm, dst_hbm, acc,
                  srcv0, srcv1, dstv0, dstv1, rowsv0, rowsv1,
                  isem0, isem1, gsem0, gsem1, wid):
    base_row = wid * BATCHES_PER_W

    def load_idx(g, sbuf, dbuf, sem):
        r0 = base_row + g * KB
        pltpu.async_copy(src_hbm.at[pl.ds(r0, KB), :], sbuf, sem)
        pltpu.async_copy(dst_hbm.at[pl.ds(r0, KB), :], dbuf, sem)

    def wait_idx(g, sbuf, dbuf, sem):
        r0 = base_row + g * KB
        pltpu.make_async_copy(src_hbm.at[pl.ds(r0, KB), :], sbuf, sem).wait()
        pltpu.make_async_copy(dst_hbm.at[pl.ds(r0, KB), :], dbuf, sem).wait()

    def fire_gathers(sbuf, rbuf, sem):
        for j in range(KB):
            pltpu.async_copy(table.at[sbuf.at[j]],
                             rbuf.at[pl.ds(j * EB, EB), :], sem)

    def drain_gathers(sbuf, rbuf, sem):
        for j in range(KB):
            pltpu.make_async_copy(table.at[sbuf.at[j]],
                                  rbuf.at[pl.ds(j * EB, EB), :], sem).wait()

    def scat(dbuf, rbuf):
        for j in range(KB):
            pltpu.sync_copy(rbuf.at[pl.ds(j * EB, EB), :],
                            acc.at[dbuf.at[j]], add=True)

    load_idx(0, srcv0, dstv0, isem0)
    wait_idx(0, srcv0, dstv0, isem0)
    fire_gathers(srcv0, rowsv0, gsem0)
    load_idx(1, srcv1, dstv1, isem1)

    def pair(i, carry):
        g0 = 2 * i
        wait_idx(g0 + 1, srcv1, dstv1, isem1)
        fire_gathers(srcv1, rowsv1, gsem1)
        drain_gathers(srcv0, rowsv0, gsem0)
        scat(dstv0, rowsv0)
        load_idx(g0 + 2, srcv0, dstv0, isem0)
        wait_idx(g0 + 2, srcv0, dstv0, isem0)
        fire_gathers(srcv0, rowsv0, gsem0)
        drain_gathers(srcv1, rowsv1, gsem1)
        scat(dstv1, rowsv1)
        load_idx(g0 + 3, srcv1, dstv1, isem1)
        return carry

    lax.fori_loop(0, NPAIRS, pair, 0)
    drain_gathers(srcv0, rowsv0, gsem0)
    wait_idx(2 * NPAIRS + 1, srcv1, dstv1, isem1)


_AGG_SCRATCH = [
    pltpu.VMEM((KB, EB), jnp.int32),
    pltpu.VMEM((KB, EB), jnp.int32),
    pltpu.VMEM((KB, EB), jnp.int32),
    pltpu.VMEM((KB, EB), jnp.int32),
    pltpu.VMEM((KB * EB, D), _f32),
    pltpu.VMEM((KB * EB, D), _f32),
    pltpu.VMEM_SHARED((N_PAD, D), _f32),
    pltpu.SemaphoreType.DMA,
    pltpu.SemaphoreType.DMA,
    pltpu.SemaphoreType.DMA,
    pltpu.SemaphoreType.DMA,
]


@functools.partial(
    pl.kernel,
    out_type=(
        jax.ShapeDtypeStruct((NC, N_PAD, D), _f32),
        jax.ShapeDtypeStruct((NC, N_PAD, D), _f32),
    ),
    mesh=_mesh,
    scratch_types=_AGG_SCRATCH + [
        pltpu.VMEM((CH, D), _f32),
        pltpu.VMEM((CH, D), _f32),
        pltpu.VMEM((CH,), _f32),
        pltpu.VMEM((CH,), _f32),
        pltpu.VMEM((D, 16), _f32),
        pltpu.VMEM((16,), _f32),
    ],
    compiler_params=pltpu.CompilerParams(use_tc_tiling_on_sc=False, needs_layout_passes=False),
)
def _sc_prep_agg1(degp_hbm, x8_hbm, src_hbm, dst_hbm, w1d_hbm,
                  p1_hbm, hs1_hbm,
                  srcv0, srcv1, dstv0, dstv1, rowsv0, rowsv1, acc,
                  isem0, isem1, gsem0, gsem1,
                  xt, ht, pd0, pd1, wdt, dvb):
    c = lax.axis_index("c")
    s = lax.axis_index("s")
    wid = c * NS + s
    rs = s * ROWS_PER_SUB
    iota, col, half = _iotas()

    pltpu.sync_copy(w1d_hbm, wdt)
    for ch in range(NCH):
        r0 = rs + ch * CH
        pltpu.sync_copy(x8_hbm.at[pl.ds(r0, CH), :], xt)
        pltpu.sync_copy(degp_hbm.at[0, pl.ds(r0, CH)], pd0)
        pltpu.sync_copy(degp_hbm.at[1, pl.ds(r0, CH)], pd1)

        def grp(g, carry):
            deg = pd0[pl.ds(g * 16, 16)] + pd1[pl.ds(g * 16, 16)] + 1.0
            dvb[...] = _rsqrt16(deg)
            for k in range(8):
                ridx = g * 16 + 2 * k + half
                h = _diag_matmul(xt, ridx, wdt, iota)
                dv8 = plsc.load_gather(dvb, [2 * k + half])
                plsc.store_scatter(ht, [ridx, col], h * dv8)
            return carry

        lax.fori_loop(0, CH // 16, grp, 0)
        pltpu.sync_copy(ht, hs1_hbm.at[c, pl.ds(r0, CH), :])
        pltpu.sync_copy(ht, acc.at[pl.ds(r0, CH), :])

    plsc.subcore_barrier()
    _agg_pipeline(hs1_hbm.at[c], src_hbm, dst_hbm, acc,
                  srcv0, srcv1, dstv0, dstv1, rowsv0, rowsv1,
                  isem0, isem1, gsem0, gsem1, wid)
    plsc.subcore_barrier()
    pltpu.sync_copy(acc.at[pl.ds(rs, ROWS_PER_SUB), :],
                    p1_hbm.at[c, pl.ds(rs, ROWS_PER_SUB), :])


@functools.partial(
    pl.kernel,
    out_type=(
        jax.ShapeDtypeStruct((NC, N_PAD, D), _f32),
        jax.ShapeDtypeStruct((NC, N_PAD, D), _f32),
    ),
    mesh=_mesh,
    scratch_types=_AGG_SCRATCH + [
        pltpu.VMEM((CH, D), _f32),
        pltpu.VMEM((CH, D), _f32),
        pltpu.VMEM((CH, D), _f32),
        pltpu.VMEM((CH, D), _f32),
        pltpu.VMEM((CH,), _f32),
        pltpu.VMEM((CH,), _f32),
        pltpu.VMEM((D, 16), _f32),
        pltpu.VMEM((16,), _f32),
        pltpu.VMEM((16,), _f32),
    ],
    compiler_params=pltpu.CompilerParams(use_tc_tiling_on_sc=False, needs_layout_passes=False),
)
def _sc_mid_agg2(degp_hbm, p1_hbm, hs1_hbm, src_hbm, dst_hbm, w2d_hbm, b1_hbm,
                 p2_hbm, hs2_hbm,
                 srcv0, srcv1, dstv0, dstv1, rowsv0, rowsv1, acc,
                 isem0, isem1, gsem0, gsem1,
                 at0, at1, hst, ht, pd0, pd1, wdt, dvb, bt):
    c = lax.axis_index("c")
    s = lax.axis_index("s")
    wid = c * NS + s
    rs = s * ROWS_PER_SUB
    iota, col, half = _iotas()

    pltpu.sync_copy(w2d_hbm, wdt)
    pltpu.sync_copy(b1_hbm, bt)
    for ch in range(NCH):
        r0 = rs + ch * CH
        pltpu.sync_copy(p1_hbm.at[0, pl.ds(r0, CH), :], at0)
        pltpu.sync_copy(p1_hbm.at[1, pl.ds(r0, CH), :], at1)
        pltpu.sync_copy(hs1_hbm.at[c, pl.ds(r0, CH), :], hst)
        pltpu.sync_copy(degp_hbm.at[0, pl.ds(r0, CH)], pd0)
        pltpu.sync_copy(degp_hbm.at[1, pl.ds(r0, CH)], pd1)

        def grp(g, carry):
            deg = pd0[pl.ds(g * 16, 16)] + pd1[pl.ds(g * 16, 16)] + 1.0
            dvb[...] = _rsqrt16(deg)
            bv = bt[...]
            for k in range(8):
                ridx = g * 16 + 2 * k + half
                agg = (plsc.load_gather(at0, [ridx, col])
                       + plsc.load_gather(at1, [ridx, col])
                       - plsc.load_gather(hst, [ridx, col]))
                dv8 = plsc.load_gather(dvb, [2 * k + half])
                h1 = jnp.maximum(agg * dv8 + bv, 0.0)
                plsc.store_scatter(at0, [ridx, col], h1)
            for k in range(8):
                ridx = g * 16 + 2 * k + half
                h2 = _diag_matmul(at0, ridx, wdt, iota)
                dv8 = plsc.load_gather(dvb, [2 * k + half])
                plsc.store_scatter(ht, [ridx, col], h2 * dv8)
            return carry

        lax.fori_loop(0, CH // 16, grp, 0)
        pltpu.sync_copy(ht, hs2_hbm.at[c, pl.ds(r0, CH), :])
        pltpu.sync_copy(ht, acc.at[pl.ds(r0, CH), :])

    plsc.subcore_barrier()
    _agg_pipeline(hs2_hbm.at[c], src_hbm, dst_hbm, acc,
                  srcv0, srcv1, dstv0, dstv1, rowsv0, rowsv1,
                  isem0, isem1, gsem0, gsem1, wid)
    plsc.subcore_barrier()
    pltpu.sync_copy(acc.at[pl.ds(rs, ROWS_PER_SUB), :],
                    p2_hbm.at[c, pl.ds(rs, ROWS_PER_SUB), :])


@functools.partial(
    pl.kernel,
    out_type=jax.ShapeDtypeStruct((N_PAD, D), _f32),
    mesh=_mesh,
    scratch_types=[
        pltpu.VMEM((CH, D), _f32),
        pltpu.VMEM((CH, D), _f32),
        pltpu.VMEM((CH, D), _f32),
        pltpu.VMEM((CH, D), _f32),
        pltpu.VMEM((CH,), _f32),
        pltpu.VMEM((CH,), _f32),
        pltpu.VMEM((D, 16), _f32),
        pltpu.VMEM((16,), _f32),
        pltpu.VMEM((16,), _f32),
        pltpu.VMEM((16,), _f32),
    ],
    compiler_params=pltpu.CompilerParams(use_tc_tiling_on_sc=False, needs_layout_passes=False),
)
def _sc_final(degp_hbm, p2_hbm, hs2_hbm, wcd_hbm, b2_hbm, bc_hbm, out_hbm,
              at0, at1, hst, ot, pd0, pd1, wdt, dvb, bt2, btc):
    c = lax.axis_index("c")
    s = lax.axis_index("s")
    wid = c * NS + s
    rw = wid * ROWS_PER_W
    iota, col, half = _iotas()

    pltpu.sync_copy(wcd_hbm, wdt)
    pltpu.sync_copy(b2_hbm, bt2)
    pltpu.sync_copy(bc_hbm, btc)
    for ch in range(ROWS_PER_W // CH):
        r0 = rw + ch * CH
        pltpu.sync_copy(p2_hbm.at[0, pl.ds(r0, CH), :], at0)
        pltpu.sync_copy(p2_hbm.at[1, pl.ds(r0, CH), :], at1)
        pltpu.sync_copy(hs2_hbm.at[c, pl.ds(r0, CH), :], hst)
        pltpu.sync_copy(degp_hbm.at[0, pl.ds(r0, CH)], pd0)
        pltpu.sync_copy(degp_hbm.at[1, pl.ds(r0, CH)], pd1)

        def grp(g, carry):
            deg = pd0[pl.ds(g * 16, 16)] + pd1[pl.ds(g * 16, 16)] + 1.0
            dvb[...] = _rsqrt16(deg)
            bv2 = bt2[...]
            bvc = btc[...]
            for k in range(8):
                ridx = g * 16 + 2 * k + half
                agg = (plsc.load_gather(at0, [ridx, col])
                       + plsc.load_gather(at1, [ridx, col])
                       - plsc.load_gather(hst, [ridx, col]))
                dv8 = plsc.load_gather(dvb, [2 * k + half])
                h2 = jnp.maximum(agg * dv8 + bv2, 0.0)
                plsc.store_scatter(at0, [ridx, col], h2)
            for k in range(8):
                ridx = g * 16 + 2 * k + half
                o = _diag_matmul(at0, ridx, wdt, iota) + bvc
                plsc.store_scatter(ot, [ridx, col], o)
            return carry

        lax.fori_loop(0, CH // 16, grp, 0)
        pltpu.sync_copy(ot, out_hbm.at[pl.ds(r0, CH), :])


def _diagonalize(wp):
    j = jnp.arange(16) % 8
    srange = jnp.arange(8)[:, None]
    return wp[(j[None, :] + srange) % 8, j[None, :]]


def kernel(x, edge_index, W1, b1, W2, b2, Wc, bc):
    src = edge_index[0].astype(jnp.int32)
    dst = edge_index[1].astype(jnp.int32)
    n_extra = IDX_ROWS * EB - E_EDGES
    pad_idx = N_NODES + (jnp.arange(n_extra, dtype=jnp.int32) % 256)
    src_p = jnp.concatenate([src, pad_idx]).reshape(IDX_ROWS, EB)
    dst_p = jnp.concatenate([dst, pad_idx]).reshape(IDX_ROWS, EB)

    x8 = jnp.zeros((N_PAD, D), _f32).at[:N_NODES, :N_FEAT].set(x)
    w1p = jnp.zeros((D, D), _f32).at[:N_FEAT, :6].set(W1)
    w2p = jnp.zeros((D, D), _f32).at[:6, :6].set(W2)
    wcp = jnp.zeros((D, D), _f32).at[:6, :N_OUT].set(Wc)
    w1d = _diagonalize(w1p)
    w2d = _diagonalize(w2p)
    wcd = _diagonalize(wcp)
    b1t = jnp.tile(jnp.zeros((D,), _f32).at[:6].set(b1), 2)
    b2t = jnp.tile(jnp.zeros((D,), _f32).at[:6].set(b2), 2)
    bct = jnp.tile(jnp.zeros((D,), _f32).at[:N_OUT].set(bc), 2)

    degp = _sc_degree(dst_p)
    p1, hs1 = _sc_prep_agg1(degp, x8, src_p, dst_p, w1d)
    p2, hs2 = _sc_mid_agg2(degp, p1, hs1, src_p, dst_p, w2d, b1t)
    out8 = _sc_final(degp, p2, hs2, wcd, b2t, bct)
    return out8[:N_NODES, :N_OUT]
